# baseline (speedup 1.0000x reference)
import functools

import jax
import jax.numpy as jnp
from jax import lax
from jax.experimental import pallas as pl
from jax.experimental.pallas import tpu as pltpu
from jax.experimental.pallas import tpu_sc as plsc

UNITS = 128
B = 4096
N = 200

_INFO = plsc.get_sparse_core_info()
_NC = _INFO.num_cores
_NS = _INFO.num_subcores
_NW = _NC * _NS
_BPW = B // _NW
_L = 16


def _sc_gather(table, first_ids, last_ids):
    mesh = plsc.VectorSubcoreMesh(core_axis_name="c", subcore_axis_name="s")

    @functools.partial(
        pl.kernel,
        mesh=mesh,
        out_type=(
            jax.ShapeDtypeStruct((B, UNITS), jnp.float32),
            jax.ShapeDtypeStruct((B, UNITS), jnp.float32),
        ),
        scratch_types=[
            pltpu.VMEM((_BPW,), jnp.int32),
            pltpu.VMEM((_BPW,), jnp.int32),
            pltpu.VMEM((_BPW, UNITS), jnp.float32),
            pltpu.VMEM((_BPW, UNITS), jnp.float32),
            pltpu.SemaphoreType.DMA,
            pltpu.SemaphoreType.DMA,
            pltpu.SemaphoreType.DMA,
            pltpu.SemaphoreType.DMA,
        ],
    )
    def k(table_hbm, gf_hbm, gl_hbm, outf_hbm, outl_hbm,
          idxf_v, idxl_v, rowsf_v, rowsl_v, semf, seml, semof, semol):
        wid = lax.axis_index("s") * _NC + lax.axis_index("c")
        base = wid * _BPW
        cif = pltpu.async_copy(gf_hbm.at[pl.ds(base, _BPW)], idxf_v, semf)
        cil = pltpu.async_copy(gl_hbm.at[pl.ds(base, _BPW)], idxl_v, seml)
        cif.wait()
        cil.wait()
        step = lax.iota(jnp.int32, _L) * N
        for j in range(_BPW // _L):
            off = (base + j * _L) * N
            sl = pl.ds(j * _L, _L)
            idxf_v[sl] = idxf_v[sl] + step + off
            idxl_v[sl] = idxl_v[sl] + step + off
        h = _BPW // 2
        chunks = [
            (idxf_v.at[pl.ds(0, h)], rowsf_v.at[pl.ds(0, h)],
             outf_hbm.at[pl.ds(base, h)]),
            (idxl_v.at[pl.ds(0, h)], rowsl_v.at[pl.ds(0, h)],
             outl_hbm.at[pl.ds(base, h)]),
            (idxf_v.at[pl.ds(h, h)], rowsf_v.at[pl.ds(h, h)],
             outf_hbm.at[pl.ds(base + h, h)]),
            (idxl_v.at[pl.ds(h, h)], rowsl_v.at[pl.ds(h, h)],
             outl_hbm.at[pl.ds(base + h, h)]),
        ]
        gsems = [semf, seml, semf, seml]
        ssems = [semof, semol, semof, semol]
        gathers = [None] * 4
        scatters = [None] * 4
        gathers[0] = pltpu.async_copy(table_hbm.at[chunks[0][0]],
                                      chunks[0][1], gsems[0])
        gathers[1] = pltpu.async_copy(table_hbm.at[chunks[1][0]],
                                      chunks[1][1], gsems[1])
        for c in range(4):
            gathers[c].wait()
            if c + 2 < 4:
                gathers[c + 2] = pltpu.async_copy(
                    table_hbm.at[chunks[c + 2][0]], chunks[c + 2][1],
                    gsems[c + 2])
            scatters[c] = pltpu.async_copy(chunks[c][1], chunks[c][2],
                                           ssems[c])
        for c in range(4):
            scatters[c].wait()

    return k(table, first_ids, last_ids)


_BM = 4096


def _proj_body(step_ref, f_ref, l_ref, fc_ref, wph_ref, w_ref, o_ref):
    use_ph = step_ref[0] == 1
    f = jnp.where(use_ph, jnp.broadcast_to(wph_ref[0:1, :], (_BM, UNITS)),
                  f_ref[...])
    l = jnp.where(use_ph, jnp.broadcast_to(wph_ref[1:2, :], (_BM, UNITS)),
                  l_ref[...])
    acc = jnp.dot(f, w_ref[:UNITS, :], preferred_element_type=jnp.float32)
    acc += jnp.dot(l, w_ref[UNITS:, :], preferred_element_type=jnp.float32)
    o_ref[...] = fc_ref[...] + acc


def _tc_project(step_arr, first_rows, last_rows, fixed, wph, w):
    grid = (B // _BM,)
    row_spec = pl.BlockSpec((_BM, UNITS), lambda i: (i, 0))
    return pl.pallas_call(
        _proj_body,
        grid=grid,
        in_specs=[
            pl.BlockSpec(memory_space=pltpu.SMEM),
            row_spec, row_spec, row_spec,
            pl.BlockSpec((2, UNITS), lambda i: (0, 0)),
            pl.BlockSpec((2 * UNITS, UNITS), lambda i: (0, 0)),
        ],
        out_specs=row_spec,
        out_shape=jax.ShapeDtypeStruct((B, UNITS), jnp.float32),
    )(step_arr, first_rows, last_rows, fixed, wph, w)


def kernel(node_embed, fixed_context, first_node, last_node, step_count,
           W_context_placeholder, W_dense):
    table = node_embed.reshape(B * N, UNITS)
    first_ids = first_node.reshape(B).astype(jnp.int32)
    last_ids = last_node.reshape(B).astype(jnp.int32)

    first_rows, last_rows = _sc_gather(table, first_ids, last_ids)

    step_arr = jnp.asarray(step_count, jnp.int32).reshape(1)
    wph = W_context_placeholder.reshape(2, UNITS)
    fixed = fixed_context.reshape(B, UNITS)

    out = _tc_project(step_arr, first_rows, last_rows, fixed, wph, W_dense)
    return out.reshape(B, 1, UNITS)


# speedup vs baseline: 1.0193x; 1.0193x over previous
import functools

import jax
import jax.numpy as jnp
from jax import lax
from jax.experimental import pallas as pl
from jax.experimental.pallas import tpu as pltpu
from jax.experimental.pallas import tpu_sc as plsc

UNITS = 128
B = 4096
N = 200

_INFO = plsc.get_sparse_core_info()
_NC = _INFO.num_cores
_NS = _INFO.num_subcores
_NW = _NC * _NS
_BPW = B // _NW
_L = 16


def _sc_gather(table, first_ids, last_ids):
    mesh = plsc.VectorSubcoreMesh(core_axis_name="c", subcore_axis_name="s")

    @functools.partial(
        pl.kernel,
        mesh=mesh,
        out_type=(
            jax.ShapeDtypeStruct((B, UNITS), jnp.float32),
            jax.ShapeDtypeStruct((B, UNITS), jnp.float32),
        ),
        scratch_types=[
            pltpu.VMEM((_BPW,), jnp.int32),
            pltpu.VMEM((_BPW,), jnp.int32),
            pltpu.VMEM((_BPW, UNITS), jnp.float32),
            pltpu.VMEM((_BPW, UNITS), jnp.float32),
            pltpu.SemaphoreType.DMA,
            pltpu.SemaphoreType.DMA,
            pltpu.SemaphoreType.DMA,
            pltpu.SemaphoreType.DMA,
        ],
    )
    def k(table_hbm, gf_hbm, gl_hbm, outf_hbm, outl_hbm,
          idxf_v, idxl_v, rowsf_v, rowsl_v, semf, seml, semof, semol):
        wid = lax.axis_index("s") * _NC + lax.axis_index("c")
        base = wid * _BPW
        cif = pltpu.async_copy(gf_hbm.at[pl.ds(base, _BPW)], idxf_v, semf)
        cil = pltpu.async_copy(gl_hbm.at[pl.ds(base, _BPW)], idxl_v, seml)
        cif.wait()
        cil.wait()
        step = lax.iota(jnp.int32, _L) * N
        for j in range(_BPW // _L):
            off = (base + j * _L) * N
            sl = pl.ds(j * _L, _L)
            idxf_v[sl] = idxf_v[sl] + step + off
            idxl_v[sl] = idxl_v[sl] + step + off
        h = _BPW // 2
        chunks = [
            (idxf_v.at[pl.ds(0, h)], rowsf_v.at[pl.ds(0, h)],
             outf_hbm.at[pl.ds(base, h)]),
            (idxl_v.at[pl.ds(0, h)], rowsl_v.at[pl.ds(0, h)],
             outl_hbm.at[pl.ds(base, h)]),
            (idxf_v.at[pl.ds(h, h)], rowsf_v.at[pl.ds(h, h)],
             outf_hbm.at[pl.ds(base + h, h)]),
            (idxl_v.at[pl.ds(h, h)], rowsl_v.at[pl.ds(h, h)],
             outl_hbm.at[pl.ds(base + h, h)]),
        ]
        gsems = [semf, seml, semf, seml]
        ssems = [semof, semol, semof, semol]
        gathers = [None] * 4
        scatters = [None] * 4
        gathers[0] = pltpu.async_copy(table_hbm.at[chunks[0][0]],
                                      chunks[0][1], gsems[0])
        gathers[1] = pltpu.async_copy(table_hbm.at[chunks[1][0]],
                                      chunks[1][1], gsems[1])
        for c in range(4):
            gathers[c].wait()
            if c + 2 < 4:
                gathers[c + 2] = pltpu.async_copy(
                    table_hbm.at[chunks[c + 2][0]], chunks[c + 2][1],
                    gsems[c + 2])
            scatters[c] = pltpu.async_copy(chunks[c][1], chunks[c][2],
                                           ssems[c])
        for c in range(4):
            scatters[c].wait()

    return k(table, first_ids, last_ids)


_BM = 2048


def _proj_body(step_ref, f_ref, l_ref, fc_ref, wph_ref, w_ref, o_ref):
    use_ph = step_ref[0] == 1
    f = jnp.where(use_ph, jnp.broadcast_to(wph_ref[0:1, :], (_BM, UNITS)),
                  f_ref[...])
    l = jnp.where(use_ph, jnp.broadcast_to(wph_ref[1:2, :], (_BM, UNITS)),
                  l_ref[...])
    acc = jnp.dot(f, w_ref[:UNITS, :], preferred_element_type=jnp.float32)
    acc += jnp.dot(l, w_ref[UNITS:, :], preferred_element_type=jnp.float32)
    o_ref[...] = fc_ref[...] + acc


def _tc_project(step_arr, first_rows, last_rows, fixed, wph, w):
    grid = (B // _BM,)
    row_spec = pl.BlockSpec((_BM, UNITS), lambda i: (i, 0))
    return pl.pallas_call(
        _proj_body,
        grid=grid,
        in_specs=[
            pl.BlockSpec(memory_space=pltpu.SMEM),
            row_spec, row_spec, row_spec,
            pl.BlockSpec((2, UNITS), lambda i: (0, 0)),
            pl.BlockSpec((2 * UNITS, UNITS), lambda i: (0, 0)),
        ],
        out_specs=row_spec,
        out_shape=jax.ShapeDtypeStruct((B, UNITS), jnp.float32),
    )(step_arr, first_rows, last_rows, fixed, wph, w)


def kernel(node_embed, fixed_context, first_node, last_node, step_count,
           W_context_placeholder, W_dense):
    table = node_embed.reshape(B * N, UNITS)
    first_ids = first_node.reshape(B).astype(jnp.int32)
    last_ids = last_node.reshape(B).astype(jnp.int32)

    first_rows, last_rows = _sc_gather(table, first_ids, last_ids)

    step_arr = jnp.asarray(step_count, jnp.int32).reshape(1)
    wph = W_context_placeholder.reshape(2, UNITS)
    fixed = fixed_context.reshape(B, UNITS)

    out = _tc_project(step_arr, first_rows, last_rows, fixed, wph, W_dense)
    return out.reshape(B, 1, UNITS)


# speedup vs baseline: 1.0238x; 1.0043x over previous
import functools

import jax
import jax.numpy as jnp
from jax import lax
from jax.experimental import pallas as pl
from jax.experimental.pallas import tpu as pltpu
from jax.experimental.pallas import tpu_sc as plsc

UNITS = 128
B = 4096
N = 200

_INFO = plsc.get_sparse_core_info()
_NC = _INFO.num_cores
_NS = _INFO.num_subcores
_NW = _NC * _NS
_BPW = B // _NW
_L = 16


def _sc_gather(table, first_ids, last_ids):
    mesh = plsc.VectorSubcoreMesh(core_axis_name="c", subcore_axis_name="s")

    @functools.partial(
        pl.kernel,
        mesh=mesh,
        out_type=jax.ShapeDtypeStruct((B, 2 * UNITS), jnp.float32),
        scratch_types=[
            pltpu.VMEM((_BPW,), jnp.int32),
            pltpu.VMEM((_BPW,), jnp.int32),
            pltpu.VMEM((_BPW, UNITS), jnp.float32),
            pltpu.VMEM((_BPW, UNITS), jnp.float32),
            pltpu.SemaphoreType.DMA,
            pltpu.SemaphoreType.DMA,
            pltpu.SemaphoreType.DMA,
            pltpu.SemaphoreType.DMA,
        ],
    )
    def k(table_hbm, gf_hbm, gl_hbm, out_hbm,
          idxf_v, idxl_v, rowsf_v, rowsl_v, semf, seml, semof, semol):
        wid = lax.axis_index("s") * _NC + lax.axis_index("c")
        base = wid * _BPW
        cif = pltpu.async_copy(gf_hbm.at[pl.ds(base, _BPW)], idxf_v, semf)
        cil = pltpu.async_copy(gl_hbm.at[pl.ds(base, _BPW)], idxl_v, seml)
        cif.wait()
        cil.wait()
        step = lax.iota(jnp.int32, _L) * N
        for j in range(_BPW // _L):
            off = (base + j * _L) * N
            sl = pl.ds(j * _L, _L)
            idxf_v[sl] = idxf_v[sl] + step + off
            idxl_v[sl] = idxl_v[sl] + step + off
        h = _BPW // 2
        chunks = [
            (idxf_v.at[pl.ds(0, h)], rowsf_v.at[pl.ds(0, h)],
             out_hbm.at[pl.ds(base, h), pl.ds(0, UNITS)]),
            (idxl_v.at[pl.ds(0, h)], rowsl_v.at[pl.ds(0, h)],
             out_hbm.at[pl.ds(base, h), pl.ds(UNITS, UNITS)]),
            (idxf_v.at[pl.ds(h, h)], rowsf_v.at[pl.ds(h, h)],
             out_hbm.at[pl.ds(base + h, h), pl.ds(0, UNITS)]),
            (idxl_v.at[pl.ds(h, h)], rowsl_v.at[pl.ds(h, h)],
             out_hbm.at[pl.ds(base + h, h), pl.ds(UNITS, UNITS)]),
        ]
        gsems = [semf, seml, semf, seml]
        ssems = [semof, semol, semof, semol]
        gathers = [None] * 4
        scatters = [None] * 4
        gathers[0] = pltpu.async_copy(table_hbm.at[chunks[0][0]],
                                      chunks[0][1], gsems[0])
        gathers[1] = pltpu.async_copy(table_hbm.at[chunks[1][0]],
                                      chunks[1][1], gsems[1])
        for c in range(4):
            gathers[c].wait()
            if c + 2 < 4:
                gathers[c + 2] = pltpu.async_copy(
                    table_hbm.at[chunks[c + 2][0]], chunks[c + 2][1],
                    gsems[c + 2])
            scatters[c] = pltpu.async_copy(chunks[c][1], chunks[c][2],
                                           ssems[c])
        for c in range(4):
            scatters[c].wait()

    return k(table, first_ids, last_ids)


_BM = 2048


def _proj_body(step_ref, cat_ref, fc_ref, wph_ref, w_ref, o_ref):
    use_ph = step_ref[0] == 1
    cat = jnp.where(use_ph,
                    jnp.broadcast_to(wph_ref[...], (_BM, 2 * UNITS)),
                    cat_ref[...])
    acc = jnp.dot(cat, w_ref[...], preferred_element_type=jnp.float32)
    o_ref[...] = fc_ref[...] + acc


def _tc_project(step_arr, cat_rows, fixed, wph, w):
    grid = (B // _BM,)
    cat_spec = pl.BlockSpec((_BM, 2 * UNITS), lambda i: (i, 0))
    row_spec = pl.BlockSpec((_BM, UNITS), lambda i: (i, 0))
    return pl.pallas_call(
        _proj_body,
        grid=grid,
        in_specs=[
            pl.BlockSpec(memory_space=pltpu.SMEM),
            cat_spec, row_spec,
            pl.BlockSpec((1, 2 * UNITS), lambda i: (0, 0)),
            pl.BlockSpec((2 * UNITS, UNITS), lambda i: (0, 0)),
        ],
        out_specs=row_spec,
        out_shape=jax.ShapeDtypeStruct((B, UNITS), jnp.float32),
    )(step_arr, cat_rows, fixed, wph, w)


def kernel(node_embed, fixed_context, first_node, last_node, step_count,
           W_context_placeholder, W_dense):
    table = node_embed.reshape(B * N, UNITS)
    first_ids = first_node.reshape(B).astype(jnp.int32)
    last_ids = last_node.reshape(B).astype(jnp.int32)

    cat_rows = _sc_gather(table, first_ids, last_ids)

    step_arr = jnp.asarray(step_count, jnp.int32).reshape(1)
    wph = W_context_placeholder.reshape(1, 2 * UNITS)
    fixed = fixed_context.reshape(B, UNITS)

    out = _tc_project(step_arr, cat_rows, fixed, wph, W_dense)
    return out.reshape(B, 1, UNITS)
